# Initial kernel scaffold; baseline (speedup 1.0000x reference)
import jax
import jax.numpy as jnp
from jax.experimental import pallas as pl


def kernel(x, edge_index, batch, W1, b1, g1, be1, m1, v1, W2, b2, g2, be2, m2, v2, fcW, fcb):
    raise NotImplementedError("write your pallas kernel here")


# speedup vs baseline: 16.6210x; 16.6210x over previous
import functools

import jax
import jax.numpy as jnp
from jax import lax
from jax.experimental import pallas as pl
from jax.experimental.pallas import tpu as pltpu, tpu_sc as plsc

N = 50000
E = 800000
D = 128
H = 64
HH = 32
C = 2
G = 128
EPS = 1e-5

NP = 50176
EP = 802816
EROWS = EP // 128
TROWS = EROWS // 16
RPT = NP // 16
BLK = 512
GRID = NP // BLK

_mesh = plsc.VectorSubcoreMesh(
    core_axis_name="c", subcore_axis_name="s", num_cores=2, num_subcores=16)

_f32 = jnp.float32
_i32 = jnp.int32


def _deg_body(dst_r, dega_o, degb_o, idx4, ones_v, zb, deg_sh):
  c = lax.axis_index("c")
  s = lax.axis_index("s")

  for i in range(8):
    ones_v[pl.ds(i * 16, 16)] = jnp.ones((16,), _f32)

  def zrow(i, carry):
    zb[pl.ds(i * 16, 16)] = jnp.zeros((16,), _f32)
    return carry
  lax.fori_loop(0, RPT // 16, zrow, 0)

  base_r = s * RPT
  pltpu.sync_copy(zb, deg_sh.at[pl.ds(base_r, RPT)])
  plsc.subcore_barrier()

  ebase = c * (EROWS // 2) + s * (TROWS // 2)
  def grp(g, carry):
    r0 = ebase + g * 4
    pltpu.sync_copy(dst_r.at[pl.ds(r0, 4), :], idx4)
    for j in range(4):
      pltpu.sync_copy(ones_v, deg_sh.at[idx4.at[j]], add=True)
    return carry
  lax.fori_loop(0, (TROWS // 2) // 4, grp, 0)

  plsc.subcore_barrier()

  pltpu.sync_copy(deg_sh.at[pl.ds(base_r, RPT)], zb)

  @pl.when(c == 0)
  def _():
    pltpu.sync_copy(zb, dega_o.at[pl.ds(base_r, RPT)])

  @pl.when(c == 1)
  def _():
    pltpu.sync_copy(zb, degb_o.at[pl.ds(base_r, RPT)])


_sc_deg = functools.partial(
    pl.kernel,
    out_type=(jax.ShapeDtypeStruct((NP,), _f32),
              jax.ShapeDtypeStruct((NP,), _f32)),
    mesh=_mesh,
    compiler_params=pltpu.CompilerParams(use_tc_tiling_on_sc=False),
    scratch_types=(
        pltpu.VMEM((4, 128), _i32),
        pltpu.VMEM((128,), _f32),
        pltpu.VMEM((RPT,), _f32),
        pltpu.VMEM_SHARED((NP,), _f32),
    ),
)(_deg_body)


def _scat_body(src_r, dst_r, y0_r, y1_r, acc0_o, acc1_o,
               idx_s, idx_d, rows, zbuf, acc_sh):
  c = lax.axis_index("c")
  s = lax.axis_index("s")

  def zrow(i, carry):
    zbuf[i, pl.ds(0, 16)] = jnp.zeros((16,), _f32)
    zbuf[i, pl.ds(16, 16)] = jnp.zeros((16,), _f32)
    return carry
  lax.fori_loop(0, TROWS, zrow, 0)

  base_r = s * RPT
  for k in range(8):
    pltpu.sync_copy(zbuf, acc_sh.at[pl.ds(base_r + k * TROWS, TROWS), :])
  plsc.subcore_barrier()

  def do_edges(table_r):
    ebase = s * TROWS
    def grp(g, carry):
      r0 = ebase + g * 8
      pltpu.sync_copy(src_r.at[pl.ds(r0, 8), :], idx_s)
      pltpu.sync_copy(dst_r.at[pl.ds(r0, 8), :], idx_d)
      for j in range(8):
        pltpu.sync_copy(table_r.at[idx_s.at[j]], rows)
        pltpu.sync_copy(rows, acc_sh.at[idx_d.at[j]], add=True)
      return carry
    lax.fori_loop(0, TROWS // 8, grp, 0)

  @pl.when(c == 0)
  def _():
    do_edges(y0_r)

  @pl.when(c == 1)
  def _():
    do_edges(y1_r)

  plsc.subcore_barrier()

  @pl.when(c == 0)
  def _():
    pltpu.sync_copy(acc_sh.at[pl.ds(base_r, RPT), :],
                    acc0_o.at[pl.ds(base_r, RPT), :])

  @pl.when(c == 1)
  def _():
    pltpu.sync_copy(acc_sh.at[pl.ds(base_r, RPT), :],
                    acc1_o.at[pl.ds(base_r, RPT), :])


_sc_scatter = functools.partial(
    pl.kernel,
    out_type=(jax.ShapeDtypeStruct((NP, HH), _f32),
              jax.ShapeDtypeStruct((NP, HH), _f32)),
    mesh=_mesh,
    compiler_params=pltpu.CompilerParams(use_tc_tiling_on_sc=False),
    scratch_types=(
        pltpu.VMEM((8, 128), _i32),
        pltpu.VMEM((8, 128), _i32),
        pltpu.VMEM((128, HH), _f32),
        pltpu.VMEM((TROWS, HH), _f32),
        pltpu.VMEM_SHARED((NP, HH), _f32),
    ),
)(_scat_body)


def _k1_body(x_r, dega_r, degb_r, wa_r, wb_r, y0_o, y1_o):
  deg = dega_r[...] + degb_r[...] + 1.0
  dinv = lax.rsqrt(deg)
  x = x_r[...]
  y0_o[...] = jnp.dot(x, wa_r[...], preferred_element_type=_f32) * dinv
  y1_o[...] = jnp.dot(x, wb_r[...], preferred_element_type=_f32) * dinv


def _k1(x, dega2, degb2, wa, wb):
  return pl.pallas_call(
      _k1_body,
      grid=(GRID,),
      in_specs=[
          pl.BlockSpec((BLK, D), lambda i: (i, 0)),
          pl.BlockSpec((BLK, 1), lambda i: (i, 0)),
          pl.BlockSpec((BLK, 1), lambda i: (i, 0)),
          pl.BlockSpec((D, HH), lambda i: (0, 0)),
          pl.BlockSpec((D, HH), lambda i: (0, 0)),
      ],
      out_specs=[
          pl.BlockSpec((BLK, HH), lambda i: (i, 0)),
          pl.BlockSpec((BLK, HH), lambda i: (i, 0)),
      ],
      out_shape=[
          jax.ShapeDtypeStruct((NP, HH), _f32),
          jax.ShapeDtypeStruct((NP, HH), _f32),
      ],
  )(x, dega2, degb2, wa, wb)


def _k2_body(acc0_r, acc1_r, y0_r, y1_r, dega_r, degb_r,
             c1a_r, c1b_r, q00_r, q01_r, q10_r, q11_r, y2a_o, y2b_o):
  deg = dega_r[...] + degb_r[...] + 1.0
  dinv = lax.rsqrt(deg)
  h1a = jnp.maximum(dinv * (acc0_r[...] + y0_r[...]) + c1a_r[...], 0.0)
  h1b = jnp.maximum(dinv * (acc1_r[...] + y1_r[...]) + c1b_r[...], 0.0)
  za = (jnp.dot(h1a, q00_r[...], preferred_element_type=_f32)
        + jnp.dot(h1b, q10_r[...], preferred_element_type=_f32))
  zb = (jnp.dot(h1a, q01_r[...], preferred_element_type=_f32)
        + jnp.dot(h1b, q11_r[...], preferred_element_type=_f32))
  y2a_o[...] = za * dinv
  y2b_o[...] = zb * dinv


def _k2(acc0, acc1, y0, y1, dega2, degb2, c1a, c1b, q00, q01, q10, q11):
  col = pl.BlockSpec((BLK, HH), lambda i: (i, 0))
  sca = pl.BlockSpec((BLK, 1), lambda i: (i, 0))
  row = pl.BlockSpec((1, HH), lambda i: (0, 0))
  wq = pl.BlockSpec((HH, HH), lambda i: (0, 0))
  return pl.pallas_call(
      _k2_body,
      grid=(GRID,),
      in_specs=[col, col, col, col, sca, sca, row, row, wq, wq, wq, wq],
      out_specs=[col, col],
      out_shape=[
          jax.ShapeDtypeStruct((NP, HH), _f32),
          jax.ShapeDtypeStruct((NP, HH), _f32),
      ],
  )(acc0, acc1, y0, y1, dega2, degb2, c1a, c1b, q00, q01, q10, q11)


def _k3_body(acc0_r, acc1_r, y2a_r, y2b_r, dega_r, degb_r,
             c2a_r, c2b_r, batch_r, sums0_o, sums1_o, cnt_o):
  i = pl.program_id(0)
  deg = dega_r[...] + degb_r[...] + 1.0
  dinv = lax.rsqrt(deg)
  h2a = jnp.maximum(dinv * (acc0_r[...] + y2a_r[...]) + c2a_r[...], 0.0)
  h2b = jnp.maximum(dinv * (acc1_r[...] + y2b_r[...]) + c2b_r[...], 0.0)

  rowid = lax.broadcasted_iota(_i32, (BLK, 1), 0) + i * BLK
  valid = rowid < N
  h2a = jnp.where(valid, h2a, 0.0)
  h2b = jnp.where(valid, h2b, 0.0)
  gid = lax.broadcasted_iota(_i32, (BLK, G), 1)
  onehot = jnp.where((batch_r[...] == gid) & valid, 1.0, 0.0)

  dn = (((0,), (0,)), ((), ()))
  ps0 = lax.dot_general(onehot, h2a, dn, preferred_element_type=_f32)
  ps1 = lax.dot_general(onehot, h2b, dn, preferred_element_type=_f32)
  ones = jnp.ones((BLK, 1), _f32)
  pc = lax.dot_general(onehot, ones, dn, preferred_element_type=_f32)

  @pl.when(i == 0)
  def _():
    sums0_o[...] = ps0
    sums1_o[...] = ps1
    cnt_o[...] = pc

  @pl.when(i != 0)
  def _():
    sums0_o[...] += ps0
    sums1_o[...] += ps1
    cnt_o[...] += pc


def _k3(acc0, acc1, y2a, y2b, dega2, degb2, c2a, c2b, batch2d):
  col = pl.BlockSpec((BLK, HH), lambda i: (i, 0))
  sca = pl.BlockSpec((BLK, 1), lambda i: (i, 0))
  row = pl.BlockSpec((1, HH), lambda i: (0, 0))
  return pl.pallas_call(
      _k3_body,
      grid=(GRID,),
      in_specs=[col, col, col, col, sca, sca, row, row,
                pl.BlockSpec((BLK, 1), lambda i: (i, 0))],
      out_specs=[
          pl.BlockSpec((G, HH), lambda i: (0, 0)),
          pl.BlockSpec((G, HH), lambda i: (0, 0)),
          pl.BlockSpec((G, 1), lambda i: (0, 0)),
      ],
      out_shape=[
          jax.ShapeDtypeStruct((G, HH), _f32),
          jax.ShapeDtypeStruct((G, HH), _f32),
          jax.ShapeDtypeStruct((G, 1), _f32),
      ],
  )(acc0, acc1, y2a, y2b, dega2, degb2, c2a, c2b, batch2d)


def _k4_body(sums0_r, sums1_r, cnt_r, fa_r, fb_r, fcb_r, out_o):
  cnt = jnp.maximum(cnt_r[...], 1.0)
  hg0 = sums0_r[...] / cnt
  hg1 = sums1_r[...] / cnt
  logits = (jnp.dot(hg0, fa_r[...], preferred_element_type=_f32)
            + jnp.dot(hg1, fb_r[...], preferred_element_type=_f32)
            + fcb_r[...])
  m = jnp.max(logits, axis=1, keepdims=True)
  e = jnp.exp(logits - m)
  lse = m + jnp.log(jnp.sum(e, axis=1, keepdims=True))
  out_o[...] = logits - lse


def _k4(sums0, sums1, cnt, fa, fb, fcb2):
  return pl.pallas_call(
      _k4_body,
      out_shape=jax.ShapeDtypeStruct((G, C), _f32),
  )(sums0, sums1, cnt, fa, fb, fcb2)


@jax.jit
def kernel(x, edge_index, batch, W1, b1, g1, be1, m1, v1,
           W2, b2, g2, be2, m2, v2, fcW, fcb):
  src = edge_index[0]
  dst = edge_index[1]
  pad = jnp.full((EP - E,), N, dtype=_i32)
  src2d = jnp.concatenate([src, pad]).reshape(EROWS, 128)
  dst2d = jnp.concatenate([dst, pad]).reshape(EROWS, 128)

  s1 = g1 / jnp.sqrt(v1 + EPS)
  W1f = W1 * s1[None, :]
  c1 = ((b1 - m1) * s1 + be1).reshape(1, H)
  s2 = g2 / jnp.sqrt(v2 + EPS)
  W2f = W2 * s2[None, :]
  c2 = ((b2 - m2) * s2 + be2).reshape(1, H)

  dega, degb = _sc_deg(dst2d)
  dega2 = dega.reshape(NP, 1)
  degb2 = degb.reshape(NP, 1)

  y0, y1 = _k1(x, dega2, degb2, W1f[:, :HH], W1f[:, HH:])
  acc0, acc1 = _sc_scatter(src2d, dst2d, y0, y1)
  y2a, y2b = _k2(acc0, acc1, y0, y1, dega2, degb2,
                 c1[:, :HH], c1[:, HH:],
                 W2f[:HH, :HH], W2f[:HH, HH:], W2f[HH:, :HH], W2f[HH:, HH:])
  acc2_0, acc2_1 = _sc_scatter(src2d, dst2d, y2a, y2b)
  sums0, sums1, cnt = _k3(acc2_0, acc2_1, y2a, y2b, dega2, degb2,
                          c2[:, :HH], c2[:, HH:], batch.reshape(N, 1))
  return _k4(sums0, sums1, cnt, fcW[:HH], fcW[HH:], fcb.reshape(1, C))


# speedup vs baseline: 28.9089x; 1.7393x over previous
import functools

import jax
import jax.numpy as jnp
from jax import lax
from jax.experimental import pallas as pl
from jax.experimental.pallas import tpu as pltpu, tpu_sc as plsc

N = 50000
E = 800000
D = 128
H = 64
HH = 32
C = 2
G = 128
EPS = 1e-5

NP = 50176
SROWS = E // 128
RPT = NP // 16
BLK = 512
GRID = NP // BLK

TR16, XT16 = 390, 10
TR32, XT32 = 195, 10

_mesh = plsc.VectorSubcoreMesh(
    core_axis_name="c", subcore_axis_name="s", num_cores=2, num_subcores=16)

_f32 = jnp.float32
_i32 = jnp.int32


def _deg_body(e2d_r, dega_o, degb_o, idx, ones_v, zb, sem, deg_sh):
  c = lax.axis_index("c")
  s = lax.axis_index("s")
  w = c * 16 + s

  for i in range(8):
    ones_v[pl.ds(i * 16, 16)] = jnp.ones((16,), _f32)

  def zrow(i, carry):
    zb[pl.ds(i * 16, 16)] = jnp.zeros((16,), _f32)
    return carry
  lax.fori_loop(0, RPT // 16, zrow, 0)

  base_r = s * RPT
  pltpu.sync_copy(zb, deg_sh.at[pl.ds(base_r, RPT)])

  cnt = TR32 + (w < XT32).astype(_i32)
  base = w * TR32 + jnp.minimum(w, XT32)
  pltpu.sync_copy(e2d_r.at[pl.ds(SROWS + base, TR32)], idx.at[pl.ds(0, TR32)])

  @pl.when(w < XT32)
  def _():
    pltpu.sync_copy(e2d_r.at[pl.ds(SROWS + base + TR32, 1)],
                    idx.at[pl.ds(TR32, 1)])

  plsc.subcore_barrier()

  def enq(r, carry):
    pltpu.async_copy(ones_v, deg_sh.at[idx.at[r]], sem, add=True)
    return carry
  lax.fori_loop(0, cnt, enq, 0)

  def drain(r, carry):
    pltpu.make_async_copy(ones_v, deg_sh.at[idx.at[r]], sem).wait()
    return carry
  lax.fori_loop(0, cnt, drain, 0)

  plsc.subcore_barrier()

  pltpu.sync_copy(deg_sh.at[pl.ds(base_r, RPT)], zb)

  @pl.when(c == 0)
  def _():
    pltpu.sync_copy(zb, dega_o.at[pl.ds(base_r, RPT)])

  @pl.when(c == 1)
  def _():
    pltpu.sync_copy(zb, degb_o.at[pl.ds(base_r, RPT)])


_sc_deg = functools.partial(
    pl.kernel,
    out_type=(jax.ShapeDtypeStruct((NP,), _f32),
              jax.ShapeDtypeStruct((NP,), _f32)),
    mesh=_mesh,
    compiler_params=pltpu.CompilerParams(use_tc_tiling_on_sc=False),
    scratch_types=(
        pltpu.VMEM((TR32 + 1, 128), _i32),
        pltpu.VMEM((128,), _f32),
        pltpu.VMEM((RPT,), _f32),
        pltpu.SemaphoreType.DMA,
        pltpu.VMEM_SHARED((NP,), _f32),
    ),
)(_deg_body)


_NBUF = 4


def _scat_body(e2d_r, y0_r, y1_r, acc0_o, acc1_o,
               idx_s, idx_d, r0b, r1b, r2b, r3b,
               s0, s1, s2, s3, acc_sh):
  c = lax.axis_index("c")
  s = lax.axis_index("s")
  rows = (r0b, r1b, r2b, r3b)
  sems = (s0, s1, s2, s3)

  def zrow(i, carry):
    r0b[i, pl.ds(0, 16)] = jnp.zeros((16,), _f32)
    r0b[i, pl.ds(16, 16)] = jnp.zeros((16,), _f32)
    return carry
  lax.fori_loop(0, 128, zrow, 0)

  base_r = s * RPT
  for k in range(RPT // 128):
    pltpu.async_copy(r0b, acc_sh.at[pl.ds(base_r + k * 128, 128), :], s1)
  pltpu.async_copy(r0b.at[pl.ds(0, RPT - 128 * (RPT // 128)), :],
                   acc_sh.at[pl.ds(base_r + 128 * (RPT // 128),
                                   RPT - 128 * (RPT // 128)), :], s1)
  for k in range(RPT // 128):
    pltpu.make_async_copy(r0b, acc_sh.at[pl.ds(base_r + k * 128, 128), :],
                          s1).wait()
  pltpu.make_async_copy(r0b.at[pl.ds(0, RPT - 128 * (RPT // 128)), :],
                        acc_sh.at[pl.ds(base_r + 128 * (RPT // 128),
                                        RPT - 128 * (RPT // 128)), :],
                        s1).wait()

  cnt = TR16 + (s < XT16).astype(_i32)
  base = s * TR16 + jnp.minimum(s, XT16)

  plsc.subcore_barrier()

  def do_edges(table_r):
    def grp(g, carry):
      gb = 16 * g
      pltpu.sync_copy(e2d_r.at[pl.ds(base + gb, 16)], idx_s)
      pltpu.sync_copy(e2d_r.at[pl.ds(SROWS + base + gb, 16)], idx_d)
      for b in range(_NBUF):
        pltpu.async_copy(table_r.at[idx_s.at[b]], rows[b], sems[b])
      for p in range(4):
        for b in range(_NBUF):
          j = 4 * p + b
          pltpu.make_async_copy(table_r.at[idx_s.at[j]], rows[b],
                                sems[b]).wait()
          pltpu.sync_copy(rows[b], acc_sh.at[idx_d.at[j]], add=True)
          if j + _NBUF < 16:
            pltpu.async_copy(table_r.at[idx_s.at[j + _NBUF]], rows[b],
                             sems[b])
      return carry
    lax.fori_loop(0, TR16 // 16, grp, 0)

    tb = 24 * 16
    pltpu.sync_copy(e2d_r.at[pl.ds(base + tb, 4)], idx_s.at[pl.ds(0, 4)])
    pltpu.sync_copy(e2d_r.at[pl.ds(base + tb + 4, 2)], idx_s.at[pl.ds(4, 2)])
    pltpu.sync_copy(e2d_r.at[pl.ds(SROWS + base + tb, 4)],
                    idx_d.at[pl.ds(0, 4)])
    pltpu.sync_copy(e2d_r.at[pl.ds(SROWS + base + tb + 4, 2)],
                    idx_d.at[pl.ds(4, 2)])

    @pl.when(cnt > TR16)
    def _():
      pltpu.sync_copy(e2d_r.at[pl.ds(base + tb + 6, 1)],
                      idx_s.at[pl.ds(6, 1)])
      pltpu.sync_copy(e2d_r.at[pl.ds(SROWS + base + tb + 6, 1)],
                      idx_d.at[pl.ds(6, 1)])

    for b in range(_NBUF):
      pltpu.async_copy(table_r.at[idx_s.at[b]], rows[b], sems[b])
    for j in range(7):
      b = j % _NBUF
      @pl.when(tb + j < cnt)
      def _():
        pltpu.make_async_copy(table_r.at[idx_s.at[j]], rows[b],
                              sems[b]).wait()
        pltpu.sync_copy(rows[b], acc_sh.at[idx_d.at[j]], add=True)
      if j + _NBUF < 7:
        @pl.when(tb + j + _NBUF < cnt)
        def _():
          pltpu.async_copy(table_r.at[idx_s.at[j + _NBUF]], rows[b], sems[b])

  @pl.when(c == 0)
  def _():
    do_edges(y0_r)

  @pl.when(c == 1)
  def _():
    do_edges(y1_r)

  plsc.subcore_barrier()

  @pl.when(c == 0)
  def _():
    pltpu.sync_copy(acc_sh.at[pl.ds(base_r, RPT), :],
                    acc0_o.at[pl.ds(base_r, RPT), :])

  @pl.when(c == 1)
  def _():
    pltpu.sync_copy(acc_sh.at[pl.ds(base_r, RPT), :],
                    acc1_o.at[pl.ds(base_r, RPT), :])


_sc_scatter = functools.partial(
    pl.kernel,
    out_type=(jax.ShapeDtypeStruct((NP, HH), _f32),
              jax.ShapeDtypeStruct((NP, HH), _f32)),
    mesh=_mesh,
    compiler_params=pltpu.CompilerParams(use_tc_tiling_on_sc=False),
    scratch_types=(
        pltpu.VMEM((16, 128), _i32),
        pltpu.VMEM((16, 128), _i32),
        pltpu.VMEM((128, HH), _f32),
        pltpu.VMEM((128, HH), _f32),
        pltpu.VMEM((128, HH), _f32),
        pltpu.VMEM((128, HH), _f32),
        pltpu.SemaphoreType.DMA,
        pltpu.SemaphoreType.DMA,
        pltpu.SemaphoreType.DMA,
        pltpu.SemaphoreType.DMA,
        pltpu.VMEM_SHARED((NP, HH), _f32),
    ),
)(_scat_body)


def _k1_body(x_r, dega_r, degb_r, wa_r, wb_r, y0_o, y1_o):
  deg = dega_r[...] + degb_r[...] + 1.0
  dinv = lax.rsqrt(deg)
  x = x_r[...]
  y0_o[...] = jnp.dot(x, wa_r[...], preferred_element_type=_f32) * dinv
  y1_o[...] = jnp.dot(x, wb_r[...], preferred_element_type=_f32) * dinv


def _k1(x, dega2, degb2, wa, wb):
  return pl.pallas_call(
      _k1_body,
      grid=(GRID,),
      in_specs=[
          pl.BlockSpec((BLK, D), lambda i: (i, 0)),
          pl.BlockSpec((BLK, 1), lambda i: (i, 0)),
          pl.BlockSpec((BLK, 1), lambda i: (i, 0)),
          pl.BlockSpec((D, HH), lambda i: (0, 0)),
          pl.BlockSpec((D, HH), lambda i: (0, 0)),
      ],
      out_specs=[
          pl.BlockSpec((BLK, HH), lambda i: (i, 0)),
          pl.BlockSpec((BLK, HH), lambda i: (i, 0)),
      ],
      out_shape=[
          jax.ShapeDtypeStruct((NP, HH), _f32),
          jax.ShapeDtypeStruct((NP, HH), _f32),
      ],
  )(x, dega2, degb2, wa, wb)


def _k2_body(acc0_r, acc1_r, y0_r, y1_r, dega_r, degb_r,
             c1a_r, c1b_r, q00_r, q01_r, q10_r, q11_r, y2a_o, y2b_o):
  deg = dega_r[...] + degb_r[...] + 1.0
  dinv = lax.rsqrt(deg)
  h1a = jnp.maximum(dinv * (acc0_r[...] + y0_r[...]) + c1a_r[...], 0.0)
  h1b = jnp.maximum(dinv * (acc1_r[...] + y1_r[...]) + c1b_r[...], 0.0)
  za = (jnp.dot(h1a, q00_r[...], preferred_element_type=_f32)
        + jnp.dot(h1b, q10_r[...], preferred_element_type=_f32))
  zb = (jnp.dot(h1a, q01_r[...], preferred_element_type=_f32)
        + jnp.dot(h1b, q11_r[...], preferred_element_type=_f32))
  y2a_o[...] = za * dinv
  y2b_o[...] = zb * dinv


def _k2(acc0, acc1, y0, y1, dega2, degb2, c1a, c1b, q00, q01, q10, q11):
  col = pl.BlockSpec((BLK, HH), lambda i: (i, 0))
  sca = pl.BlockSpec((BLK, 1), lambda i: (i, 0))
  row = pl.BlockSpec((1, HH), lambda i: (0, 0))
  wq = pl.BlockSpec((HH, HH), lambda i: (0, 0))
  return pl.pallas_call(
      _k2_body,
      grid=(GRID,),
      in_specs=[col, col, col, col, sca, sca, row, row, wq, wq, wq, wq],
      out_specs=[col, col],
      out_shape=[
          jax.ShapeDtypeStruct((NP, HH), _f32),
          jax.ShapeDtypeStruct((NP, HH), _f32),
      ],
  )(acc0, acc1, y0, y1, dega2, degb2, c1a, c1b, q00, q01, q10, q11)


def _k3_body(acc0_r, acc1_r, y2a_r, y2b_r, dega_r, degb_r,
             c2a_r, c2b_r, batch_r, sums0_o, sums1_o, cnt_o):
  i = pl.program_id(0)
  deg = dega_r[...] + degb_r[...] + 1.0
  dinv = lax.rsqrt(deg)
  h2a = jnp.maximum(dinv * (acc0_r[...] + y2a_r[...]) + c2a_r[...], 0.0)
  h2b = jnp.maximum(dinv * (acc1_r[...] + y2b_r[...]) + c2b_r[...], 0.0)

  rowid = lax.broadcasted_iota(_i32, (BLK, 1), 0) + i * BLK
  valid = rowid < N
  h2a = jnp.where(valid, h2a, 0.0)
  h2b = jnp.where(valid, h2b, 0.0)
  gid = lax.broadcasted_iota(_i32, (BLK, G), 1)
  onehot = jnp.where((batch_r[...] == gid) & valid, 1.0, 0.0)

  dn = (((0,), (0,)), ((), ()))
  ps0 = lax.dot_general(onehot, h2a, dn, preferred_element_type=_f32)
  ps1 = lax.dot_general(onehot, h2b, dn, preferred_element_type=_f32)
  ones = jnp.ones((BLK, 1), _f32)
  pc = lax.dot_general(onehot, ones, dn, preferred_element_type=_f32)

  @pl.when(i == 0)
  def _():
    sums0_o[...] = ps0
    sums1_o[...] = ps1
    cnt_o[...] = pc

  @pl.when(i != 0)
  def _():
    sums0_o[...] += ps0
    sums1_o[...] += ps1
    cnt_o[...] += pc


def _k3(acc0, acc1, y2a, y2b, dega2, degb2, c2a, c2b, batch2d):
  col = pl.BlockSpec((BLK, HH), lambda i: (i, 0))
  sca = pl.BlockSpec((BLK, 1), lambda i: (i, 0))
  row = pl.BlockSpec((1, HH), lambda i: (0, 0))
  return pl.pallas_call(
      _k3_body,
      grid=(GRID,),
      in_specs=[col, col, col, col, sca, sca, row, row,
                pl.BlockSpec((BLK, 1), lambda i: (i, 0))],
      out_specs=[
          pl.BlockSpec((G, HH), lambda i: (0, 0)),
          pl.BlockSpec((G, HH), lambda i: (0, 0)),
          pl.BlockSpec((G, 1), lambda i: (0, 0)),
      ],
      out_shape=[
          jax.ShapeDtypeStruct((G, HH), _f32),
          jax.ShapeDtypeStruct((G, HH), _f32),
          jax.ShapeDtypeStruct((G, 1), _f32),
      ],
  )(acc0, acc1, y2a, y2b, dega2, degb2, c2a, c2b, batch2d)


def _k4_body(sums0_r, sums1_r, cnt_r, fa_r, fb_r, fcb_r, out_o):
  cnt = jnp.maximum(cnt_r[...], 1.0)
  hg0 = sums0_r[...] / cnt
  hg1 = sums1_r[...] / cnt
  logits = (jnp.dot(hg0, fa_r[...], preferred_element_type=_f32)
            + jnp.dot(hg1, fb_r[...], preferred_element_type=_f32)
            + fcb_r[...])
  m = jnp.max(logits, axis=1, keepdims=True)
  e = jnp.exp(logits - m)
  lse = m + jnp.log(jnp.sum(e, axis=1, keepdims=True))
  out_o[...] = logits - lse


def _k4(sums0, sums1, cnt, fa, fb, fcb2):
  return pl.pallas_call(
      _k4_body,
      out_shape=jax.ShapeDtypeStruct((G, C), _f32),
  )(sums0, sums1, cnt, fa, fb, fcb2)


@jax.jit
def kernel(x, edge_index, batch, W1, b1, g1, be1, m1, v1,
           W2, b2, g2, be2, m2, v2, fcW, fcb):
  e2d = edge_index.reshape(2 * SROWS, 128)

  s1 = g1 / jnp.sqrt(v1 + EPS)
  W1f = W1 * s1[None, :]
  c1 = ((b1 - m1) * s1 + be1).reshape(1, H)
  s2 = g2 / jnp.sqrt(v2 + EPS)
  W2f = W2 * s2[None, :]
  c2 = ((b2 - m2) * s2 + be2).reshape(1, H)

  dega, degb = _sc_deg(e2d)
  dega2 = dega.reshape(NP, 1)
  degb2 = degb.reshape(NP, 1)

  y0, y1 = _k1(x, dega2, degb2, W1f[:, :HH], W1f[:, HH:])
  acc0, acc1 = _sc_scatter(e2d, y0, y1)
  y2a, y2b = _k2(acc0, acc1, y0, y1, dega2, degb2,
                 c1[:, :HH], c1[:, HH:],
                 W2f[:HH, :HH], W2f[:HH, HH:], W2f[HH:, :HH], W2f[HH:, HH:])
  acc2_0, acc2_1 = _sc_scatter(e2d, y2a, y2b)
  sums0, sums1, cnt = _k3(acc2_0, acc2_1, y2a, y2b, dega2, degb2,
                          c2[:, :HH], c2[:, HH:], batch.reshape(N, 1))
  return _k4(sums0, sums1, cnt, fcW[:HH], fcW[HH:], fcb.reshape(1, C))
